# baseline (speedup 1.0000x reference)
import jax
import jax.numpy as jnp
from jax.experimental import pallas as pl
from jax.experimental.pallas import tpu as pltpu

_NUM_EXPERTS = 8
_TOP_K = 2
_ROW_TILE = 256


def _router_kernel(x_ref, wr_ref, idx_ref, w_ref, pos_ref, sizes_ref):
    x = x_ref[...]
    wr = wr_ref[...]
    T = x.shape[0]
    logits = jax.lax.dot_general(
        wr, x, (((1,), (1,)), ((), ())), preferred_element_type=jnp.float32
    )
    neg_inf = jnp.float32(-jnp.inf)
    e1 = jnp.argmax(logits, axis=0).astype(jnp.int32)
    m1 = jnp.max(logits, axis=0)
    rows = jax.lax.broadcasted_iota(jnp.int32, logits.shape, 0)
    masked = jnp.where(rows == e1[None, :], neg_inf, logits)
    e2 = jnp.argmax(masked, axis=0).astype(jnp.int32)
    m2 = jnp.max(masked, axis=0)
    t = jnp.exp(m2 - m1)
    w_ref[...] = jnp.stack([1.0 / (1.0 + t), t / (1.0 + t)], axis=0)
    idx_ref[...] = jnp.stack([e1, e2], axis=0)

    oh1 = (rows == e1[None, :]).astype(jnp.bfloat16)
    oh2 = (rows == e2[None, :]).astype(jnp.bfloat16)
    ti = jax.lax.broadcasted_iota(jnp.int32, (T, T), 0)
    tj = jax.lax.broadcasted_iota(jnp.int32, (T, T), 1)
    le = (ti <= tj).astype(jnp.bfloat16)
    c1 = jax.lax.dot_general(oh1, le, (((1,), (0,)), ((), ())),
                             preferred_element_type=jnp.float32)
    c2 = jax.lax.dot_general(oh2, le, (((1,), (0,)), ((), ())),
                             preferred_element_type=jnp.float32)
    oh1f = oh1.astype(jnp.float32)
    oh2f = oh2.astype(jnp.float32)
    c1ex = c1 - oh1f
    c2ex = c2 - oh2f
    rank0 = jnp.sum(oh1f * (c1ex + c2ex), axis=0)
    rank1 = jnp.sum(oh2f * (c1 + c2ex), axis=0)
    sizes = c1[:, T - 1] + c2[:, T - 1]
    ei = jax.lax.broadcasted_iota(jnp.int32, (8, 8), 0)
    ej = jax.lax.broadcasted_iota(jnp.int32, (8, 8), 1)
    lt8 = (ei < ej).astype(jnp.float32)
    starts = jax.lax.dot_general(sizes[None, :], lt8, (((1,), (0,)), ((), ())),
                                 preferred_element_type=jnp.float32,
                                 precision=jax.lax.Precision.HIGHEST)[0]
    s0 = jnp.sum(oh1f * starts[:, None], axis=0)
    s1 = jnp.sum(oh2f * starts[:, None], axis=0)
    pos_ref[...] = jnp.stack([s0 + rank0, s1 + rank1]).astype(jnp.int32)
    sizes_ref[...] = sizes[None, :].astype(jnp.int32)


def _router(x, w_router):
    T = x.shape[0]
    return pl.pallas_call(
        _router_kernel,
        out_shape=[
            jax.ShapeDtypeStruct((2, T), jnp.int32),
            jax.ShapeDtypeStruct((2, T), jnp.float32),
            jax.ShapeDtypeStruct((2, T), jnp.int32),
            jax.ShapeDtypeStruct((1, _NUM_EXPERTS), jnp.int32),
        ],
    )(x, w_router)


def _gmm_kernel(tile_ref, exp_ref, fv_ref, x_ref, scale_ref, win_ref,
                wout_ref, out_ref):
    i = pl.program_id(0)
    x = x_ref[...].astype(jnp.bfloat16)
    scale = scale_ref[0, 0, :]
    gh = jnp.dot(x, win_ref[0].astype(jnp.bfloat16),
                 preferred_element_type=jnp.float32)
    ff = gh.shape[1] // 2
    gate = gh[:, :ff]
    up = gh[:, ff:]
    h = gate * jax.lax.logistic(gate) * up
    y = jnp.dot(h.astype(jnp.bfloat16), wout_ref[0].astype(jnp.bfloat16),
                preferred_element_type=jnp.float32)
    y = y * scale[:, None]

    @pl.when(fv_ref[i] == 1)
    def _init():
        out_ref[...] = jnp.zeros_like(out_ref)

    out_ref[...] += y


def _grouped_mlp(x_sorted, scale3, w_in, w_out, tile_ids, exp_ids, fv):
    m, d = x_sorted.shape
    e, _, ff2 = w_in.shape
    ff = ff2 // 2
    b = _ROW_TILE
    g_max = tile_ids.shape[0]
    grid_spec = pltpu.PrefetchScalarGridSpec(
        num_scalar_prefetch=3,
        grid=(g_max,),
        in_specs=[
            pl.BlockSpec((b, d), lambda i, t, ex, f: (t[i], 0)),
            pl.BlockSpec((1, 1, b), lambda i, t, ex, f: (i, 0, 0)),
            pl.BlockSpec((1, d, ff2), lambda i, t, ex, f: (ex[i], 0, 0)),
            pl.BlockSpec((1, ff, d), lambda i, t, ex, f: (ex[i], 0, 0)),
        ],
        out_specs=pl.BlockSpec((b, d), lambda i, t, ex, f: (t[i], 0)),
    )
    return pl.pallas_call(
        _gmm_kernel,
        grid_spec=grid_spec,
        out_shape=jax.ShapeDtypeStruct((m, d), jnp.float32),
        compiler_params=pltpu.CompilerParams(
            dimension_semantics=("arbitrary",)),
    )(tile_ids, exp_ids, fv, x_sorted, scale3, w_in, w_out)


def kernel(layer_input, W_router, W_in, W_out):
    bsz, seq, d = layer_input.shape
    x = layer_input.reshape(-1, d)
    T = x.shape[0]
    M = _TOP_K * T
    B = _ROW_TILE
    num_tiles = M // B
    g_max = num_tiles + _NUM_EXPERTS - 1

    idx2, w2, pos2, sizes2 = _router(x, W_router)

    tok_sorted = jnp.zeros((M,), jnp.int32).at[pos2.reshape(-1)].set(
        jnp.concatenate([jnp.arange(T, dtype=jnp.int32)] * 2))

    sizes = sizes2[0]
    ends = jnp.cumsum(sizes)
    starts = ends - sizes
    first_tile = starts // B
    last_tile = jnp.where(sizes > 0, (ends - 1) // B, first_tile)
    tiles_g = jnp.where(sizes > 0, last_tile - first_tile + 1, 0)
    wends = jnp.cumsum(tiles_g)
    wstart = wends - tiles_g
    total = wends[-1]
    iarr = jnp.arange(g_max, dtype=jnp.int32)
    g = (jnp.searchsorted(wstart, iarr, side="right") - 1).astype(jnp.int32)
    g = jnp.clip(g, 0, _NUM_EXPERTS - 1)
    valid = iarr < total
    tile_ids = jnp.where(
        valid,
        jnp.clip(first_tile[g] + (iarr - wstart[g]), 0, num_tiles - 1),
        num_tiles - 1,
    ).astype(jnp.int32)
    exp_ids = g
    fv = jnp.concatenate(
        [jnp.ones((1,), jnp.int32),
         (tile_ids[1:] != tile_ids[:-1]).astype(jnp.int32)])

    row_idx = tile_ids[:, None] * B + jnp.arange(B, dtype=jnp.int32)[None, :]
    e_row = jnp.sum(ends[None, None, :] <= row_idx[:, :, None],
                    axis=-1).astype(jnp.int32)
    scale = jnp.where((e_row == exp_ids[:, None]) & valid[:, None],
                      1.0, 0.0).astype(jnp.float32)
    scale3 = scale.reshape(g_max, 1, B)

    x_sorted = x[tok_sorted]
    y_sorted = _grouped_mlp(x_sorted, scale3, W_in, W_out,
                            tile_ids, exp_ids, fv)
    out = (w2[0][:, None] * y_sorted[pos2[0]]
           + w2[1][:, None] * y_sorted[pos2[1]])
    return out.reshape(bsz, seq, d)


# speedup vs baseline: 1.0299x; 1.0299x over previous
import jax
import jax.numpy as jnp
from jax.experimental import pallas as pl
from jax.experimental.pallas import tpu as pltpu

_NUM_EXPERTS = 8
_TOP_K = 2
_ROW_TILE = 256


def _router_kernel(x_ref, wr_ref, idx_ref, w_ref, pos_ref, sizes_ref):
    x = x_ref[...]
    wr = wr_ref[...]
    T = x.shape[0]
    logits = jax.lax.dot_general(
        wr, x, (((1,), (1,)), ((), ())), preferred_element_type=jnp.float32
    )
    neg_inf = jnp.float32(-jnp.inf)
    e1 = jnp.argmax(logits, axis=0).astype(jnp.int32)
    m1 = jnp.max(logits, axis=0)
    rows = jax.lax.broadcasted_iota(jnp.int32, logits.shape, 0)
    masked = jnp.where(rows == e1[None, :], neg_inf, logits)
    e2 = jnp.argmax(masked, axis=0).astype(jnp.int32)
    m2 = jnp.max(masked, axis=0)
    t = jnp.exp(m2 - m1)
    w_ref[...] = jnp.stack([1.0 / (1.0 + t), t / (1.0 + t)], axis=0)
    idx_ref[...] = jnp.stack([e1, e2], axis=0)

    oh1 = (rows == e1[None, :]).astype(jnp.bfloat16)
    oh2 = (rows == e2[None, :]).astype(jnp.bfloat16)
    ti = jax.lax.broadcasted_iota(jnp.int32, (T, T), 0)
    tj = jax.lax.broadcasted_iota(jnp.int32, (T, T), 1)
    le = (ti <= tj).astype(jnp.bfloat16)
    c1 = jax.lax.dot_general(oh1, le, (((1,), (0,)), ((), ())),
                             preferred_element_type=jnp.float32)
    c2 = jax.lax.dot_general(oh2, le, (((1,), (0,)), ((), ())),
                             preferred_element_type=jnp.float32)
    oh1f = oh1.astype(jnp.float32)
    oh2f = oh2.astype(jnp.float32)
    c1ex = c1 - oh1f
    c2ex = c2 - oh2f
    rank0 = jnp.sum(oh1f * (c1ex + c2ex), axis=0)
    rank1 = jnp.sum(oh2f * (c1 + c2ex), axis=0)
    sizes = c1[:, T - 1] + c2[:, T - 1]
    ei = jax.lax.broadcasted_iota(jnp.int32, (8, 8), 0)
    ej = jax.lax.broadcasted_iota(jnp.int32, (8, 8), 1)
    lt8 = (ei < ej).astype(jnp.float32)
    starts = jax.lax.dot_general(sizes[None, :], lt8, (((1,), (0,)), ((), ())),
                                 preferred_element_type=jnp.float32,
                                 precision=jax.lax.Precision.HIGHEST)[0]
    s0 = jnp.sum(oh1f * starts[:, None], axis=0)
    s1 = jnp.sum(oh2f * starts[:, None], axis=0)
    pos_ref[...] = jnp.stack([s0 + rank0, s1 + rank1]).astype(jnp.int32)
    sizes_ref[...] = sizes[None, :].astype(jnp.int32)


def _router(x, w_router):
    T = x.shape[0]
    return pl.pallas_call(
        _router_kernel,
        out_shape=[
            jax.ShapeDtypeStruct((2, T), jnp.int32),
            jax.ShapeDtypeStruct((2, T), jnp.float32),
            jax.ShapeDtypeStruct((2, T), jnp.int32),
            jax.ShapeDtypeStruct((1, _NUM_EXPERTS), jnp.int32),
        ],
    )(x, w_router)


def _gmm_kernel(tile_ref, exp_ref, fv_ref, x_ref, scale_ref, win_ref,
                wout_ref, out_ref):
    i = pl.program_id(0)
    x = x_ref[...]
    scale = scale_ref[0, 0, :]
    gh = jnp.dot(x, win_ref[0].astype(jnp.bfloat16),
                 preferred_element_type=jnp.float32)
    ff = gh.shape[1] // 2
    gate = gh[:, :ff]
    up = gh[:, ff:]
    h = gate * jax.lax.logistic(gate) * up
    y = jnp.dot(h.astype(jnp.bfloat16), wout_ref[0].astype(jnp.bfloat16),
                preferred_element_type=jnp.float32)
    y = y * scale[:, None]

    @pl.when(fv_ref[i] == 1)
    def _init():
        out_ref[...] = jnp.zeros_like(out_ref)

    out_ref[...] += y


def _grouped_mlp(x_sorted, scale3, w_in, w_out, tile_ids, exp_ids, fv):
    m, d = x_sorted.shape
    e, _, ff2 = w_in.shape
    ff = ff2 // 2
    b = _ROW_TILE
    g_max = tile_ids.shape[0]
    grid_spec = pltpu.PrefetchScalarGridSpec(
        num_scalar_prefetch=3,
        grid=(g_max,),
        in_specs=[
            pl.BlockSpec((b, d), lambda i, t, ex, f: (t[i], 0)),
            pl.BlockSpec((1, 1, b), lambda i, t, ex, f: (i, 0, 0)),
            pl.BlockSpec((1, d, ff2), lambda i, t, ex, f: (ex[i], 0, 0)),
            pl.BlockSpec((1, ff, d), lambda i, t, ex, f: (ex[i], 0, 0)),
        ],
        out_specs=pl.BlockSpec((b, d), lambda i, t, ex, f: (t[i], 0)),
    )
    return pl.pallas_call(
        _gmm_kernel,
        grid_spec=grid_spec,
        out_shape=jax.ShapeDtypeStruct((m, d), jnp.float32),
        compiler_params=pltpu.CompilerParams(
            dimension_semantics=("arbitrary",)),
    )(tile_ids, exp_ids, fv, x_sorted, scale3, w_in, w_out)


def kernel(layer_input, W_router, W_in, W_out):
    bsz, seq, d = layer_input.shape
    x = layer_input.reshape(-1, d)
    T = x.shape[0]
    M = _TOP_K * T
    B = _ROW_TILE
    num_tiles = M // B
    g_max = num_tiles + _NUM_EXPERTS - 1

    idx2, w2, pos2, sizes2 = _router(x, W_router)

    tok_sorted = jnp.zeros((M,), jnp.int32).at[pos2.reshape(-1)].set(
        jnp.concatenate([jnp.arange(T, dtype=jnp.int32)] * 2))

    sizes = sizes2[0]
    ends = jnp.cumsum(sizes)
    starts = ends - sizes
    first_tile = starts // B
    last_tile = jnp.where(sizes > 0, (ends - 1) // B, first_tile)
    tiles_g = jnp.where(sizes > 0, last_tile - first_tile + 1, 0)
    wends = jnp.cumsum(tiles_g)
    wstart = wends - tiles_g
    total = wends[-1]
    iarr = jnp.arange(g_max, dtype=jnp.int32)
    g = (jnp.searchsorted(wstart, iarr, side="right") - 1).astype(jnp.int32)
    g = jnp.clip(g, 0, _NUM_EXPERTS - 1)
    valid = iarr < total
    tile_ids = jnp.where(
        valid,
        jnp.clip(first_tile[g] + (iarr - wstart[g]), 0, num_tiles - 1),
        num_tiles - 1,
    ).astype(jnp.int32)
    exp_ids = g
    fv = jnp.concatenate(
        [jnp.ones((1,), jnp.int32),
         (tile_ids[1:] != tile_ids[:-1]).astype(jnp.int32)])

    row_idx = tile_ids[:, None] * B + jnp.arange(B, dtype=jnp.int32)[None, :]
    e_row = jnp.sum(ends[None, None, :] <= row_idx[:, :, None],
                    axis=-1).astype(jnp.int32)
    scale = jnp.where((e_row == exp_ids[:, None]) & valid[:, None],
                      1.0, 0.0).astype(jnp.float32)
    scale3 = scale.reshape(g_max, 1, B)

    x_sorted = x.astype(jnp.bfloat16)[tok_sorted]
    y_sorted = _grouped_mlp(x_sorted, scale3, W_in, W_out,
                            tile_ids, exp_ids, fv)
    y2 = y_sorted[pos2.reshape(-1)]
    out = w2[0][:, None] * y2[:T] + w2[1][:, None] * y2[T:]
    return out.reshape(bsz, seq, d)
